# Initial kernel scaffold; baseline (speedup 1.0000x reference)
import jax
import jax.numpy as jnp
from jax.experimental import pallas as pl


def kernel(user, item_p, item_n, mask, users_int, users_pop, items_int, items_pop):
    raise NotImplementedError("write your pallas kernel here")


# speedup vs baseline: 1.2867x; 1.2867x over previous
import functools

import jax
import jax.numpy as jnp
from jax import lax
from jax.experimental import pallas as pl
from jax.experimental.pallas import tpu as pltpu
from jax.experimental.pallas import tpu_sc as plsc

B = 16384
L = 20
N = B * L
D = 64
NUM_E = 1000000
INT_WEIGHT = 0.1
POP_WEIGHT = 0.1
DIS_PEN = 0.01

NC = 2
NS = 16
NW = NC * NS
NP = N // NW
C = 128
NCHUNK = NP // C

RS = (N // 128, 128)

_mesh = plsc.VectorSubcoreMesh(
    core_axis_name="c", subcore_axis_name="s", num_cores=NC, num_subcores=NS
)


def _iota16():
  return lax.iota(jnp.int32, 16)


_GDN = lax.GatherDimensionNumbers(
    offset_dims=(), collapsed_slice_dims=(0,), start_index_map=(0,))


def _lane_perm(x, perm):
  return lax.gather(x, perm[:, None], _GDN, slice_sizes=(1,),
                    mode=lax.GatherScatterMode.PROMISE_IN_BOUNDS)


def _hsum(x):
  iota = _iota16()
  for sh in (8, 4, 2, 1):
    x = x + _lane_perm(x, iota ^ sh)
  return x


def _dot4(a, b):
  return (a[0] * b[0] + a[1] * b[1]) + (a[2] * b[2] + a[3] * b[3])


def _sq4(a, b):
  d0 = a[0] - b[0]
  d1 = a[1] - b[1]
  d2 = a[2] - b[2]
  d3 = a[3] - b[3]
  return (d0 * d0 + d1 * d1) + (d2 * d2 + d3 * d3)


@functools.partial(
    pl.kernel,
    out_type=(
        jax.ShapeDtypeStruct((N,), jnp.float32),
        jax.ShapeDtypeStruct((N,), jnp.float32),
        jax.ShapeDtypeStruct((N,), jnp.float32),
        jax.ShapeDtypeStruct((N,), jnp.float32),
        jax.ShapeDtypeStruct((N,), jnp.float32),
        jax.ShapeDtypeStruct((N,), jnp.float32),
        jax.ShapeDtypeStruct((N,), jnp.float32),
        jax.ShapeDtypeStruct((NUM_E,), jnp.int32),
        jax.ShapeDtypeStruct((NUM_E,), jnp.int32),
    ),
    mesh=_mesh,
    compiler_params=pltpu.CompilerParams(needs_layout_passes=False,
                                         use_tc_tiling_on_sc=False),
    scratch_types=[
        pltpu.VMEM((C,), jnp.int32),
        pltpu.VMEM((C,), jnp.int32),
        pltpu.VMEM((C,), jnp.int32),
        pltpu.VMEM((C, D), jnp.float32),
        pltpu.VMEM((C, D), jnp.float32),
        pltpu.VMEM((C, D), jnp.float32),
        pltpu.VMEM((C, D), jnp.float32),
        pltpu.VMEM((C, D), jnp.float32),
        pltpu.VMEM((C, D), jnp.float32),
        pltpu.VMEM((C,), jnp.float32),
        pltpu.VMEM((C,), jnp.float32),
        pltpu.VMEM((C,), jnp.float32),
        pltpu.VMEM((C,), jnp.float32),
        pltpu.VMEM((C,), jnp.float32),
        pltpu.VMEM((C,), jnp.float32),
        pltpu.VMEM((C,), jnp.float32),
        pltpu.VMEM((C,), jnp.int32),
        pltpu.VMEM((C,), jnp.int32),
        pltpu.SemaphoreType.DMA,
        pltpu.SemaphoreType.DMA,
    ],
)
def _main_sc(uidx, pidx, nidx, t_uint, t_upop, t_iint, t_ipop,
             o_pint, o_nint, o_ppop, o_npop, o_d2p, o_d2n, o_d2u,
             o_tagi, o_tagu,
             v_ui, v_pi, v_ni,
             r_uint, r_upop, r_ipint, r_ippop, r_inint, r_inpop,
             s_pint, s_nint, s_ppop, s_npop, v_d2p, v_d2n, v_d2u,
             v_pos, v_posn, sem_g, sem_s):
  wid = lax.axis_index("s") * NC + lax.axis_index("c")
  iota = _iota16()
  lane15 = iota == 15

  def chunk_body(j, carry):
    base = wid * NP + j * C

    pltpu.sync_copy(uidx.at[pl.ds(base, C)], v_ui)
    pltpu.sync_copy(pidx.at[pl.ds(base, C)], v_pi)
    pltpu.sync_copy(nidx.at[pl.ds(base, C)], v_ni)
    cps = [
        pltpu.async_copy(t_uint.at[v_ui], r_uint, sem_g),
        pltpu.async_copy(t_upop.at[v_ui], r_upop, sem_g),
        pltpu.async_copy(t_iint.at[v_pi], r_ipint, sem_g),
        pltpu.async_copy(t_ipop.at[v_pi], r_ippop, sem_g),
        pltpu.async_copy(t_iint.at[v_ni], r_inint, sem_g),
        pltpu.async_copy(t_ipop.at[v_ni], r_inpop, sem_g),
    ]

    def pos_body(k, c):
      v = iota + jnp.full((16,), base + k * 16, jnp.int32)
      v_pos[pl.ds(k * 16, 16)] = v
      v_posn[pl.ds(k * 16, 16)] = v + N
      return c
    lax.fori_loop(0, C // 16, pos_body, 0)

    for cp in cps:
      cp.wait()

    def row_body(r, c):
      ui = [r_uint[r, pl.ds(k * 16, 16)] for k in range(4)]
      up = [r_upop[r, pl.ds(k * 16, 16)] for k in range(4)]
      pi_ = [r_ipint[r, pl.ds(k * 16, 16)] for k in range(4)]
      pp = [r_ippop[r, pl.ds(k * 16, 16)] for k in range(4)]
      ni_ = [r_inint[r, pl.ds(k * 16, 16)] for k in range(4)]
      np_ = [r_inpop[r, pl.ds(k * 16, 16)] for k in range(4)]
      ridx = jnp.full((16,), r, jnp.int32)

      def red_store(vec, ref):
        plsc.store_scatter(ref, [ridx], _hsum(vec), mask=lane15)

      red_store(_dot4(ui, pi_), s_pint)
      red_store(_dot4(ui, ni_), s_nint)
      red_store(_dot4(up, pp), s_ppop)
      red_store(_dot4(up, np_), s_npop)
      red_store(_sq4(pi_, pp), v_d2p)
      red_store(_sq4(ni_, np_), v_d2n)
      red_store(_sq4(ui, up), v_d2u)
      return c
    lax.fori_loop(0, C, row_body, 0)

    pltpu.sync_copy(s_pint, o_pint.at[pl.ds(base, C)])
    pltpu.sync_copy(s_nint, o_nint.at[pl.ds(base, C)])
    pltpu.sync_copy(s_ppop, o_ppop.at[pl.ds(base, C)])
    pltpu.sync_copy(s_npop, o_npop.at[pl.ds(base, C)])
    pltpu.sync_copy(v_d2p, o_d2p.at[pl.ds(base, C)])
    pltpu.sync_copy(v_d2n, o_d2n.at[pl.ds(base, C)])
    pltpu.sync_copy(v_d2u, o_d2u.at[pl.ds(base, C)])

    sc1 = pltpu.async_copy(v_pos, o_tagi.at[v_pi], sem_s)
    sc2 = pltpu.async_copy(v_posn, o_tagi.at[v_ni], sem_s)
    sc3 = pltpu.async_copy(v_pos, o_tagu.at[v_ui], sem_s)
    sc1.wait()
    sc2.wait()
    sc3.wait()
    return carry

  lax.fori_loop(0, NCHUNK, chunk_body, 0)


@functools.partial(
    pl.kernel,
    out_type=jax.ShapeDtypeStruct((NW * 8,), jnp.float32),
    mesh=_mesh,
    compiler_params=pltpu.CompilerParams(needs_layout_passes=False,
                                         use_tc_tiling_on_sc=False),
    scratch_types=[
        pltpu.VMEM((C,), jnp.int32),
        pltpu.VMEM((C,), jnp.int32),
        pltpu.VMEM((C,), jnp.int32),
        pltpu.VMEM((C,), jnp.int32),
        pltpu.VMEM((C,), jnp.int32),
        pltpu.VMEM((C,), jnp.int32),
        pltpu.VMEM((C,), jnp.float32),
        pltpu.VMEM((C,), jnp.float32),
        pltpu.VMEM((C,), jnp.float32),
        pltpu.VMEM((16,), jnp.float32),
        pltpu.VMEM((16,), jnp.float32),
        pltpu.VMEM((16,), jnp.float32),
        pltpu.VMEM((16,), jnp.float32),
        pltpu.VMEM((16,), jnp.float32),
        pltpu.SemaphoreType.DMA,
    ],
)
def _reduce_sc(uidx, pidx, nidx, d2p, d2n, d2u, tagi, tagu,
               o_part,
               v_ui, v_pi, v_ni, v_tu, v_tp, v_tn, v_d2p, v_d2n, v_d2u,
               a_isum, a_icnt, a_usum, a_ucnt, v_out, sem):
  wid = lax.axis_index("s") * NC + lax.axis_index("c")
  iota = _iota16()
  lane15 = iota == 15
  zeros = jnp.zeros((16,), jnp.float32)
  ones = jnp.ones((16,), jnp.float32)
  a_isum[...] = zeros
  a_icnt[...] = zeros
  a_usum[...] = zeros
  a_ucnt[...] = zeros

  def chunk_body(j, carry):
    base = wid * NP + j * C
    pltpu.sync_copy(uidx.at[pl.ds(base, C)], v_ui)
    pltpu.sync_copy(pidx.at[pl.ds(base, C)], v_pi)
    pltpu.sync_copy(nidx.at[pl.ds(base, C)], v_ni)
    pltpu.sync_copy(d2p.at[pl.ds(base, C)], v_d2p)
    pltpu.sync_copy(d2n.at[pl.ds(base, C)], v_d2n)
    pltpu.sync_copy(d2u.at[pl.ds(base, C)], v_d2u)
    c1 = pltpu.async_copy(tagu.at[v_ui], v_tu, sem)
    c2 = pltpu.async_copy(tagi.at[v_pi], v_tp, sem)
    c3 = pltpu.async_copy(tagi.at[v_ni], v_tn, sem)
    c1.wait()
    c2.wait()
    c3.wait()

    def grp_body(k, c):
      pos = iota + jnp.full((16,), base + k * 16, jnp.int32)
      sl = pl.ds(k * 16, 16)
      m_p = v_tp[sl] == pos
      m_n = v_tn[sl] == (pos + N)
      m_u = v_tu[sl] == pos
      a_isum[...] = a_isum[...] + (jnp.where(m_p, v_d2p[sl], zeros)
                                   + jnp.where(m_n, v_d2n[sl], zeros))
      a_icnt[...] = a_icnt[...] + (jnp.where(m_p, ones, zeros)
                                   + jnp.where(m_n, ones, zeros))
      a_usum[...] = a_usum[...] + jnp.where(m_u, v_d2u[sl], zeros)
      a_ucnt[...] = a_ucnt[...] + jnp.where(m_u, ones, zeros)
      return c
    lax.fori_loop(0, C // 16, grp_body, 0)
    return carry

  lax.fori_loop(0, NCHUNK, chunk_body, 0)

  v_out[...] = jnp.zeros((16,), jnp.float32)
  for slot, ref in enumerate((a_isum, a_icnt, a_usum, a_ucnt)):
    plsc.store_scatter(v_out, [jnp.full((16,), slot, jnp.int32)],
                       _hsum(ref[...]), mask=lane15)
  pltpu.sync_copy(v_out.at[pl.ds(0, 8)], o_part.at[pl.ds(wid * 8, 8)])


def _tc_loss_body(pint_ref, ppop_ref, nint_ref, npop_ref, maskf_ref,
                  part_ref, out_ref):
  pi = pint_ref[...]
  pp = ppop_ref[...]
  ni = nint_ref[...]
  np_ = npop_ref[...]
  mf = maskf_ref[...]
  nmf = 1.0 - mf

  def logsig(x):
    return jnp.log(jax.nn.sigmoid(x))

  loss_int = -jnp.mean(mf * logsig(pi - ni))
  loss_pop = (-jnp.mean(mf * logsig(np_ - pp))
              - jnp.mean(nmf * logsig(pp - np_)))
  loss_total = -jnp.mean(logsig((pi + pp) - (ni + np_)))

  ss = jnp.sum(part_ref[...], axis=0, keepdims=True)
  item_mse = ss[0, 0] / (ss[0, 1] * D)
  user_mse = ss[0, 2] / (ss[0, 3] * D)

  loss = (INT_WEIGHT * loss_int + POP_WEIGHT * loss_pop + loss_total
          - DIS_PEN * (item_mse + user_mse))
  out_ref[...] = jnp.reshape(loss, (1, 1))


def kernel(user, item_p, item_n, mask, users_int, users_pop, items_int,
           items_pop):
  uf = user.reshape(-1).astype(jnp.int32)
  pf = item_p.reshape(-1).astype(jnp.int32)
  nf = item_n.reshape(-1).astype(jnp.int32)

  (p_int, n_int, p_pop, n_pop, d2p, d2n, d2u, tagi, tagu) = _main_sc(
      uf, pf, nf, users_int, users_pop, items_int, items_pop)

  partials = _reduce_sc(uf, pf, nf, d2p, d2n, d2u, tagi, tagu)

  maskf = mask.reshape(RS).astype(jnp.float32)
  lossm = pl.pallas_call(
      _tc_loss_body,
      out_shape=jax.ShapeDtypeStruct((1, 1), jnp.float32),
  )(p_int.reshape(RS), p_pop.reshape(RS), n_int.reshape(RS),
    n_pop.reshape(RS), maskf, partials.reshape(NW, 8))
  loss = lossm[0, 0]

  return (loss, p_int.reshape(B, L), p_pop.reshape(B, L),
          n_int.reshape(B, L), n_pop.reshape(B, L))
